# baseline (speedup 1.0000x reference)
import functools

import jax
import jax.numpy as jnp
from jax import lax
from jax.experimental import pallas as pl
from jax.experimental.pallas import tpu as pltpu
from jax.experimental.pallas import tpu_sc as plsc

N = 10000
M = 5000
E = 160000
D_IN = 256
D_HID = 256
N_CLS = 40

NC = 2
NS = 16
K = 128
NB = 80
NBK = NB * K

NP = 10112
MPD = 5376
MCT = 6144
NDUM = N
MDUM = 5120
NWR = 632
MZR = 336
MSR = 384


def _make_agg(DH: int):
    mesh = plsc.VectorSubcoreMesh(core_axis_name="c", subcore_axis_name="s")
    cpr = DH // 16

    @functools.partial(
        pl.kernel,
        out_type=jax.ShapeDtypeStruct((2 * NP, DH), jnp.float32),
        mesh=mesh,
        compiler_params=pltpu.CompilerParams(needs_layout_passes=False),
        scratch_types=[
            pltpu.VMEM((K,), jnp.int32),
            pltpu.VMEM((K,), jnp.int32),
            pltpu.VMEM((K,), jnp.int32),
            pltpu.VMEM((K, DH), jnp.float32),
            pltpu.VMEM((K,), jnp.int32),
            pltpu.VMEM((K,), jnp.int32),
            pltpu.VMEM((K,), jnp.int32),
            pltpu.VMEM((K, DH), jnp.float32),
            pltpu.VMEM((MSR,), jnp.float32),
            pltpu.VMEM((K,), jnp.float32),
            pltpu.VMEM_SHARED((NP, DH), jnp.float32),
            pltpu.VMEM_SHARED((MCT,), jnp.float32),
            pltpu.SemaphoreType.DMA,
            pltpu.SemaphoreType.DMA,
            pltpu.SemaphoreType.DMA,
            pltpu.SemaphoreType.DMA,
        ],
    )
    def agg(hs, vidx, eidx, out, vb0, ig0, eb0, rb0, vb1, ig1, eb1, rb1,
            cv1, ones1, buf, cnt_sp, semA0, semA1, semI0, semI1):
        rows_buf = rb0
        set0 = (vb0, ig0, eb0, rb0, semA0, semI0)
        set1 = (vb1, ig1, eb1, rb1, semA1, semI1)
        cid = lax.axis_index("c")
        sid = lax.axis_index("s")
        off = cid * NP
        base = pl.multiple_of(sid * NBK, 8)

        z16 = jnp.zeros((16,), jnp.float32)
        o16 = jnp.ones((16,), jnp.float32)

        def body_fill(i, carry):
            for c in range(cpr):
                rows_buf[i, pl.ds(c * 16, 16)] = z16
            return carry
        lax.fori_loop(0, K, body_fill, 0)

        def body_fill1(i, carry):
            ones1[pl.ds(i * 16, 16)] = o16
            return carry
        lax.fori_loop(0, K // 16, body_fill1, 0)

        def body_fill2(i, carry):
            cv1[pl.ds(i * 16, 16)] = z16
            return carry
        lax.fori_loop(0, MSR // 16, body_fill2, 0)

        for r0 in range(0, MZR, K):
            nr = min(K, MZR - r0)
            pltpu.sync_copy(rows_buf.at[pl.ds(0, nr)],
                            buf.at[pl.ds(sid * MZR + r0, nr)])
        pltpu.sync_copy(cv1, cnt_sp.at[pl.ds(sid * MSR, MSR)])

        plsc.subcore_barrier()

        def fire_idx(b, st):
            vb, _, eb, _, _, semI = st
            s0 = pl.multiple_of(base + b * K, 8)
            pltpu.async_copy(vidx.at[pl.ds(s0, K)], vb, semI)
            pltpu.async_copy(eidx.at[pl.ds(s0, K)], eb, semI)

        def fire_gather_a(st):
            vb, ig, eb, rb, semA, semI = st
            pltpu.make_async_copy(vidx.at[pl.ds(0, K)], vb, semI).wait()
            pltpu.make_async_copy(eidx.at[pl.ds(0, K)], eb, semI).wait()
            for i in range(K // 16):
                sl = pl.ds(i * 16, 16)
                ig[sl] = vb[sl] + off
            pltpu.async_copy(hs.at[ig], rb, semA)

        def drain_a(st):
            vb, ig, eb, rb, semA, semI = st
            pltpu.make_async_copy(hs.at[pl.ds(0, K)], rb, semA).wait()
            pltpu.sync_copy(rb, buf.at[eb], add=True)
            pltpu.sync_copy(ones1, cnt_sp.at[eb], add=True)

        fire_idx(0, set0)
        fire_gather_a(set0)
        fire_idx(1, set1)

        def body_a(g, carry):
            b0 = g * 2
            fire_gather_a(set1)
            drain_a(set0)

            @pl.when(b0 + 2 < NB)
            def _():
                fire_idx(b0 + 2, set0)
            drain_a(set1)

            @pl.when(b0 + 2 < NB)
            def _():
                fire_gather_a(set0)
                fire_idx(b0 + 3, set1)
            return carry
        lax.fori_loop(0, NB // 2, body_a, 0)

        plsc.subcore_barrier()

        @pl.when(sid < MPD // MSR)
        def _scale():
            pltpu.sync_copy(cnt_sp.at[pl.ds(sid * MSR, MSR)], cv1)
            for r0 in range(0, MSR, K):
                nr = min(K, MSR - r0)
                e0 = sid * MSR + r0
                pltpu.sync_copy(buf.at[pl.ds(e0, nr)], rows_buf.at[pl.ds(0, nr)])

                def body_a2(m, carry, r0=r0):
                    cnt = plsc.load_gather(
                        cv1, [jnp.full((16,), r0 + m, jnp.int32)])
                    inv = 1.0 / jnp.maximum(cnt, 1.0)
                    for c in range(cpr):
                        sl = pl.ds(c * 16, 16)
                        rows_buf[m, sl] = rows_buf[m, sl] * inv
                    return carry
                lax.fori_loop(0, nr, body_a2, 0)
                pltpu.sync_copy(rows_buf.at[pl.ds(0, nr)],
                                out.at[pl.ds(off + e0, nr)])

        plsc.subcore_barrier()

        for r0 in range(0, NWR, K):
            nr = min(K, NWR - r0)
            a0 = sid * NWR + r0
            pltpu.sync_copy(hs.at[pl.ds(off + a0, nr)], rows_buf.at[pl.ds(0, nr)])
            pltpu.sync_copy(rows_buf.at[pl.ds(0, nr)], buf.at[pl.ds(a0, nr)])

        plsc.subcore_barrier()

        def fire_gather_b(st):
            vb, ig, eb, rb, semA, semI = st
            pltpu.make_async_copy(vidx.at[pl.ds(0, K)], vb, semI).wait()
            pltpu.make_async_copy(eidx.at[pl.ds(0, K)], eb, semI).wait()
            for i in range(K // 16):
                sl = pl.ds(i * 16, 16)
                ig[sl] = eb[sl] + off
            pltpu.async_copy(out.at[ig], rb, semA)

        def drain_b(st):
            vb, ig, eb, rb, semA, semI = st
            pltpu.make_async_copy(out.at[pl.ds(0, K)], rb, semA).wait()
            pltpu.sync_copy(rb, buf.at[vb], add=True)

        fire_idx(0, set0)
        fire_gather_b(set0)
        fire_idx(1, set1)

        def body_b(g, carry):
            b0 = g * 2
            fire_gather_b(set1)
            drain_b(set0)

            @pl.when(b0 + 2 < NB)
            def _():
                fire_idx(b0 + 2, set0)
            drain_b(set1)

            @pl.when(b0 + 2 < NB)
            def _():
                fire_gather_b(set0)
                fire_idx(b0 + 3, set1)
            return carry
        lax.fori_loop(0, NB // 2, body_b, 0)

        plsc.subcore_barrier()

        for r0 in range(0, NWR, K):
            nr = min(K, NWR - r0)
            a0 = sid * NWR + r0
            pltpu.sync_copy(buf.at[pl.ds(a0, nr)], rows_buf.at[pl.ds(0, nr)])
            pltpu.sync_copy(rows_buf.at[pl.ds(0, nr)], out.at[pl.ds(off + a0, nr)])

    return agg


_agg_256 = _make_agg(128)


def _tc1_body(x_ref, w_ref, b_ref, o_ref):
    h = lax.dot_general(x_ref[...], w_ref[...], (((1,), (1,)), ((), ())),
                        preferred_element_type=jnp.float32)
    h = h + b_ref[...]
    o_ref[0] = h[:, :128]
    o_ref[1] = h[:, 128:]


def _tc2_body(x0_ref, x1_ref, w_ref, b_ref, o_ref):
    x0 = jnp.maximum(x0_ref[0], 0.0)
    x1 = jnp.maximum(x1_ref[0], 0.0)
    w = w_ref[...]
    h = lax.dot_general(x0, w[:, :128], (((1,), (1,)), ((), ())),
                        preferred_element_type=jnp.float32)
    h = h + lax.dot_general(x1, w[:, 128:], (((1,), (1,)), ((), ())),
                            preferred_element_type=jnp.float32)
    h = h + b_ref[...]
    hp = jnp.concatenate([h, jnp.zeros_like(h)], axis=1)
    o_ref[0] = hp
    o_ref[1] = hp


_RB = 2528


def _tc1(x_pad, w1, b1r):
    return pl.pallas_call(
        _tc1_body,
        grid=(NP // _RB,),
        in_specs=[
            pl.BlockSpec((_RB, D_IN), lambda i: (i, 0)),
            pl.BlockSpec((D_HID, D_IN), lambda i: (0, 0)),
            pl.BlockSpec((1, D_HID), lambda i: (0, 0)),
        ],
        out_specs=pl.BlockSpec((2, _RB, 128), lambda i: (0, i, 0)),
        out_shape=jax.ShapeDtypeStruct((2, NP, 128), jnp.float32),
    )(x_pad, w1, b1r)


def _tc2(hs1, w2p, b2r):
    return pl.pallas_call(
        _tc2_body,
        grid=(NP // _RB,),
        in_specs=[
            pl.BlockSpec((1, _RB, 128), lambda i: (0, i, 0)),
            pl.BlockSpec((1, _RB, 128), lambda i: (1, i, 0)),
            pl.BlockSpec((64, D_HID), lambda i: (0, 0)),
            pl.BlockSpec((1, 64), lambda i: (0, 0)),
        ],
        out_specs=pl.BlockSpec((2, _RB, 128), lambda i: (0, i, 0)),
        out_shape=jax.ShapeDtypeStruct((2, NP, 128), jnp.float32),
    )(hs1, hs1, w2p, b2r)


def kernel(X, W1, b1, W2, b2, v_idx, e_idx):
    x_pad = jnp.pad(X, ((0, NP - N), (0, 0)))
    vpad = jnp.pad(v_idx, (0, NS * NBK - E), constant_values=NDUM)
    epad = jnp.pad(e_idx, (0, NS * NBK - E), constant_values=MDUM)
    h3 = _tc1(x_pad, W1, b1.reshape(1, -1))
    g1 = _agg_256(h3.reshape(2 * NP, 128), vpad, epad)
    w2p = jnp.pad(W2, ((0, 64 - N_CLS), (0, 0)))
    b2r = jnp.pad(b2, (0, 64 - N_CLS)).reshape(1, -1)
    h2 = _tc2(g1.reshape(2, NP, 128), w2p, b2r)
    g2 = _agg_256(h2.reshape(2 * NP, 128), vpad, epad)
    return g2[:N, :N_CLS]


# speedup vs baseline: 1.1637x; 1.1637x over previous
import functools

import jax
import jax.numpy as jnp
from jax import lax
from jax.experimental import pallas as pl
from jax.experimental.pallas import tpu as pltpu
from jax.experimental.pallas import tpu_sc as plsc

N = 10000
M = 5000
E = 160000
D_IN = 256
D_HID = 256
N_CLS = 40

NC = 2
NS = 16
K = 160
NB = 64
NBK = NB * K

NP = 10112
MPD = 5376
MCT = 6144
NDUM = N
MDUM = 5120
NWR = 632
MZR = 336
MSR = 384


def _make_agg(DH: int, emit_cnt: bool):
    mesh = plsc.VectorSubcoreMesh(core_axis_name="c", subcore_axis_name="s")
    cpr = DH // 16

    out_sds = jax.ShapeDtypeStruct((2 * NP, DH), jnp.float32)
    cnt_sds = jax.ShapeDtypeStruct((MCT,), jnp.float32)
    scratch = [
        pltpu.VMEM((K,), jnp.int32),
        pltpu.VMEM((K,), jnp.int32),
        pltpu.VMEM((K,), jnp.int32),
        pltpu.VMEM((K, DH), jnp.float32),
        pltpu.VMEM((K,), jnp.int32),
        pltpu.VMEM((K,), jnp.int32),
        pltpu.VMEM((K,), jnp.int32),
        pltpu.VMEM((K, DH), jnp.float32),
        pltpu.VMEM((MSR,), jnp.float32),
        pltpu.VMEM((K,), jnp.float32),
        pltpu.VMEM_SHARED((NP, DH), jnp.float32),
        pltpu.VMEM_SHARED((MCT,), jnp.float32),
        pltpu.SemaphoreType.DMA,
        pltpu.SemaphoreType.DMA,
    ]
    if not emit_cnt:
        del scratch[11]

    @functools.partial(
        pl.kernel,
        out_type=(out_sds, cnt_sds) if emit_cnt else out_sds,
        mesh=mesh,
        compiler_params=pltpu.CompilerParams(needs_layout_passes=False),
        scratch_types=scratch,
    )
    def agg(*refs):
        if emit_cnt:
            (hs, vidx, eidx, out, cnt_out, vb0, ig0, eb0, rb0, vb1, ig1, eb1,
             rb1, cv1, ones1, buf, cnt_sp, semA0, semA1) = refs
        else:
            (hs, vidx, eidx, cnt_in, out, vb0, ig0, eb0, rb0, vb1, ig1, eb1,
             rb1, cv1, ones1, buf, semA0, semA1) = refs
        rows_buf = rb0
        set0 = (vb0, ig0, eb0, rb0, semA0)
        set1 = (vb1, ig1, eb1, rb1, semA1)
        cid = lax.axis_index("c")
        sid = lax.axis_index("s")
        off = cid * NP
        base = pl.multiple_of(sid * NBK, 8)

        z16 = jnp.zeros((16,), jnp.float32)
        o16 = jnp.ones((16,), jnp.float32)

        def body_fill(i, carry):
            for c in range(cpr):
                rows_buf[i, pl.ds(c * 16, 16)] = z16
            return carry
        lax.fori_loop(0, K, body_fill, 0)

        if emit_cnt:
            def body_fill1(i, carry):
                ones1[pl.ds(i * 16, 16)] = o16
                return carry
            lax.fori_loop(0, K // 16, body_fill1, 0)

            def body_fill2(i, carry):
                cv1[pl.ds(i * 16, 16)] = z16
                return carry
            lax.fori_loop(0, MSR // 16, body_fill2, 0)

        for r0 in range(0, MZR, K):
            nr = min(K, MZR - r0)
            pltpu.sync_copy(rows_buf.at[pl.ds(0, nr)],
                            buf.at[pl.ds(sid * MZR + r0, nr)])
        if emit_cnt:
            pltpu.sync_copy(cv1, cnt_sp.at[pl.ds(sid * MSR, MSR)])

        plsc.subcore_barrier()

        def fire_a(b, st):
            vb, ig, eb, rb, sem = st
            s0 = pl.multiple_of(base + b * K, 8)
            pltpu.sync_copy(vidx.at[pl.ds(s0, K)], vb)
            pltpu.sync_copy(eidx.at[pl.ds(s0, K)], eb)
            for i in range(K // 16):
                sl = pl.ds(i * 16, 16)
                ig[sl] = vb[sl] + off
            pltpu.async_copy(hs.at[ig], rb, sem)

        def drain_a(st):
            vb, ig, eb, rb, sem = st
            pltpu.make_async_copy(hs.at[pl.ds(0, K)], rb, sem).wait()
            pltpu.sync_copy(rb, buf.at[eb], add=True)
            if emit_cnt:
                pltpu.sync_copy(ones1, cnt_sp.at[eb], add=True)

        fire_a(0, set0)

        def body_a(g, carry):
            b0 = g * 2
            fire_a(b0 + 1, set1)
            drain_a(set0)

            @pl.when(b0 + 2 < NB)
            def _():
                fire_a(b0 + 2, set0)
            drain_a(set1)
            return carry
        lax.fori_loop(0, NB // 2, body_a, 0)

        plsc.subcore_barrier()

        if emit_cnt:
            pltpu.sync_copy(cnt_sp.at[pl.ds(sid * MSR, MSR)], cv1)
            pltpu.sync_copy(cv1, cnt_out.at[pl.ds(sid * MSR, MSR)])

        @pl.when(sid < MPD // MSR)
        def _scale():
            if not emit_cnt:
                pltpu.sync_copy(cnt_in.at[pl.ds(sid * MSR, MSR)], cv1)
            for r0 in range(0, MSR, K):
                nr = min(K, MSR - r0)
                e0 = sid * MSR + r0
                pltpu.sync_copy(buf.at[pl.ds(e0, nr)], rows_buf.at[pl.ds(0, nr)])

                def body_a2(m, carry, r0=r0):
                    cnt = plsc.load_gather(
                        cv1, [jnp.full((16,), r0 + m, jnp.int32)])
                    inv = 1.0 / jnp.maximum(cnt, 1.0)
                    for c in range(cpr):
                        sl = pl.ds(c * 16, 16)
                        rows_buf[m, sl] = rows_buf[m, sl] * inv
                    return carry
                lax.fori_loop(0, nr, body_a2, 0)
                pltpu.sync_copy(rows_buf.at[pl.ds(0, nr)],
                                out.at[pl.ds(off + e0, nr)])

        plsc.subcore_barrier()

        for r0 in range(0, NWR, K):
            nr = min(K, NWR - r0)
            a0 = sid * NWR + r0
            pltpu.sync_copy(hs.at[pl.ds(off + a0, nr)], rows_buf.at[pl.ds(0, nr)])
            pltpu.sync_copy(rows_buf.at[pl.ds(0, nr)], buf.at[pl.ds(a0, nr)])

        plsc.subcore_barrier()

        def fire_b(b, st):
            vb, ig, eb, rb, sem = st
            s0 = pl.multiple_of(base + b * K, 8)
            pltpu.sync_copy(vidx.at[pl.ds(s0, K)], vb)
            pltpu.sync_copy(eidx.at[pl.ds(s0, K)], eb)
            for i in range(K // 16):
                sl = pl.ds(i * 16, 16)
                ig[sl] = eb[sl] + off
            pltpu.async_copy(out.at[ig], rb, sem)

        def drain_b(st):
            vb, ig, eb, rb, sem = st
            pltpu.make_async_copy(out.at[pl.ds(0, K)], rb, sem).wait()
            pltpu.sync_copy(rb, buf.at[vb], add=True)

        fire_b(0, set0)

        def body_b(g, carry):
            b0 = g * 2
            fire_b(b0 + 1, set1)
            drain_b(set0)

            @pl.when(b0 + 2 < NB)
            def _():
                fire_b(b0 + 2, set0)
            drain_b(set1)
            return carry
        lax.fori_loop(0, NB // 2, body_b, 0)

        plsc.subcore_barrier()

        for r0 in range(0, NWR, K):
            nr = min(K, NWR - r0)
            a0 = sid * NWR + r0
            pltpu.sync_copy(buf.at[pl.ds(a0, nr)], rows_buf.at[pl.ds(0, nr)])
            pltpu.sync_copy(rows_buf.at[pl.ds(0, nr)], out.at[pl.ds(off + a0, nr)])

    return agg


_agg_emit = _make_agg(128, emit_cnt=True)
_agg_use = _make_agg(128, emit_cnt=False)


def _tc1_body(x_ref, w_ref, b_ref, o_ref):
    h = lax.dot_general(x_ref[...], w_ref[...], (((1,), (1,)), ((), ())),
                        preferred_element_type=jnp.float32)
    h = h + b_ref[...]
    o_ref[0] = h[:, :128]
    o_ref[1] = h[:, 128:]


def _tc2_body(x0_ref, x1_ref, w_ref, b_ref, o_ref):
    x0 = jnp.maximum(x0_ref[0], 0.0)
    x1 = jnp.maximum(x1_ref[0], 0.0)
    w = w_ref[...]
    h = lax.dot_general(x0, w[:, :128], (((1,), (1,)), ((), ())),
                        preferred_element_type=jnp.float32)
    h = h + lax.dot_general(x1, w[:, 128:], (((1,), (1,)), ((), ())),
                            preferred_element_type=jnp.float32)
    h = h + b_ref[...]
    hp = jnp.concatenate([h, jnp.zeros_like(h)], axis=1)
    o_ref[0] = hp
    o_ref[1] = hp


_RB = 2528


def _tc1(x_pad, w1, b1r):
    return pl.pallas_call(
        _tc1_body,
        grid=(NP // _RB,),
        in_specs=[
            pl.BlockSpec((_RB, D_IN), lambda i: (i, 0)),
            pl.BlockSpec((D_HID, D_IN), lambda i: (0, 0)),
            pl.BlockSpec((1, D_HID), lambda i: (0, 0)),
        ],
        out_specs=pl.BlockSpec((2, _RB, 128), lambda i: (0, i, 0)),
        out_shape=jax.ShapeDtypeStruct((2, NP, 128), jnp.float32),
    )(x_pad, w1, b1r)


def _tc2(hs1, w2p, b2r):
    return pl.pallas_call(
        _tc2_body,
        grid=(NP // _RB,),
        in_specs=[
            pl.BlockSpec((1, _RB, 128), lambda i: (0, i, 0)),
            pl.BlockSpec((1, _RB, 128), lambda i: (1, i, 0)),
            pl.BlockSpec((64, D_HID), lambda i: (0, 0)),
            pl.BlockSpec((1, 64), lambda i: (0, 0)),
        ],
        out_specs=pl.BlockSpec((2, _RB, 128), lambda i: (0, i, 0)),
        out_shape=jax.ShapeDtypeStruct((2, NP, 128), jnp.float32),
    )(hs1, hs1, w2p, b2r)


def kernel(X, W1, b1, W2, b2, v_idx, e_idx):
    x_pad = jnp.pad(X, ((0, NP - N), (0, 0)))
    vpad = jnp.pad(v_idx, (0, NS * NBK - E), constant_values=NDUM)
    epad = jnp.pad(e_idx, (0, NS * NBK - E), constant_values=MDUM)
    h3 = _tc1(x_pad, W1, b1.reshape(1, -1))
    g1, cnt1 = _agg_emit(h3.reshape(2 * NP, 128), vpad, epad)
    w2p = jnp.pad(W2, ((0, 64 - N_CLS), (0, 0)))
    b2r = jnp.pad(b2, (0, 64 - N_CLS)).reshape(1, -1)
    h2 = _tc2(g1.reshape(2, NP, 128), w2p, b2r)
    g2 = _agg_use(h2.reshape(2 * NP, 128), vpad, epad, cnt1)
    return g2[:N, :N_CLS]


# speedup vs baseline: 1.4077x; 1.2096x over previous
import functools

import jax
import jax.numpy as jnp
from jax import lax
from jax.experimental import pallas as pl
from jax.experimental.pallas import tpu as pltpu
from jax.experimental.pallas import tpu_sc as plsc

N = 10000
M = 5000
E = 160000
D_IN = 256
D_HID = 256
N_CLS = 40

NC = 2
NS = 16
K = 160
NB = 64
NBK = NB * K

NP = 10112
MPD = 5376
MCT = 6144
NDUM = N
MDUM = 5120
NWR = 632
MZR = 336
MSR = 384


def _make_agg(DH: int, emit_cnt: bool, tct: bool = True):
    mesh = plsc.VectorSubcoreMesh(core_axis_name="c", subcore_axis_name="s")
    cpr = DH // 16

    out_sds = jax.ShapeDtypeStruct((2 * NP, DH), jnp.float32)
    cnt_sds = jax.ShapeDtypeStruct((MCT,), jnp.float32)
    scratch = [
        pltpu.VMEM((K,), jnp.int32),
        pltpu.VMEM((K,), jnp.int32),
        pltpu.VMEM((K,), jnp.int32),
        pltpu.VMEM((K, DH), jnp.float32),
        pltpu.VMEM((K,), jnp.int32),
        pltpu.VMEM((K,), jnp.int32),
        pltpu.VMEM((K,), jnp.int32),
        pltpu.VMEM((K, DH), jnp.float32),
        pltpu.VMEM((MSR,), jnp.float32),
        pltpu.VMEM((K,), jnp.float32),
        pltpu.VMEM_SHARED((NP, DH), jnp.float32),
        pltpu.VMEM_SHARED((MCT,), jnp.float32),
        pltpu.SemaphoreType.DMA,
        pltpu.SemaphoreType.DMA,
    ]
    if not emit_cnt:
        del scratch[11]

    @functools.partial(
        pl.kernel,
        out_type=(out_sds, cnt_sds) if emit_cnt else out_sds,
        mesh=mesh,
        compiler_params=pltpu.CompilerParams(
            needs_layout_passes=False, use_tc_tiling_on_sc=tct),
        scratch_types=scratch,
    )
    def agg(*refs):
        if emit_cnt:
            (hs, vidx, eidx, out, cnt_out, vb0, ig0, eb0, rb0, vb1, ig1, eb1,
             rb1, cv1, ones1, buf, cnt_sp, semA0, semA1) = refs
        else:
            (hs, vidx, eidx, cnt_in, out, vb0, ig0, eb0, rb0, vb1, ig1, eb1,
             rb1, cv1, ones1, buf, semA0, semA1) = refs
        rows_buf = rb0
        set0 = (vb0, ig0, eb0, rb0, semA0)
        set1 = (vb1, ig1, eb1, rb1, semA1)
        cid = lax.axis_index("c")
        sid = lax.axis_index("s")
        off = cid * NP
        base = pl.multiple_of(sid * NBK, 8)

        z16 = jnp.zeros((16,), jnp.float32)
        o16 = jnp.ones((16,), jnp.float32)

        def body_fill(i, carry):
            for c in range(cpr):
                rows_buf[i, pl.ds(c * 16, 16)] = z16
            return carry
        lax.fori_loop(0, K, body_fill, 0)

        if emit_cnt:
            def body_fill1(i, carry):
                ones1[pl.ds(i * 16, 16)] = o16
                return carry
            lax.fori_loop(0, K // 16, body_fill1, 0)

            def body_fill2(i, carry):
                cv1[pl.ds(i * 16, 16)] = z16
                return carry
            lax.fori_loop(0, MSR // 16, body_fill2, 0)

        for r0 in range(0, MZR, K):
            nr = min(K, MZR - r0)
            pltpu.sync_copy(rows_buf.at[pl.ds(0, nr)],
                            buf.at[pl.ds(sid * MZR + r0, nr)])
        if emit_cnt:
            pltpu.sync_copy(cv1, cnt_sp.at[pl.ds(sid * MSR, MSR)])

        plsc.subcore_barrier()

        def fire_a(b, st):
            vb, ig, eb, rb, sem = st
            s0 = pl.multiple_of(base + b * K, 8)
            pltpu.sync_copy(vidx.at[pl.ds(s0, K)], vb)
            pltpu.sync_copy(eidx.at[pl.ds(s0, K)], eb)
            for i in range(K // 16):
                sl = pl.ds(i * 16, 16)
                ig[sl] = vb[sl] + off
            pltpu.async_copy(hs.at[ig], rb, sem)

        def drain_a(st):
            vb, ig, eb, rb, sem = st
            pltpu.make_async_copy(hs.at[pl.ds(0, K)], rb, sem).wait()
            pltpu.sync_copy(rb, buf.at[eb], add=True)
            if emit_cnt:
                pltpu.sync_copy(ones1, cnt_sp.at[eb], add=True)

        fire_a(0, set0)

        def body_a(g, carry):
            b0 = g * 2
            fire_a(b0 + 1, set1)
            drain_a(set0)

            @pl.when(b0 + 2 < NB)
            def _():
                fire_a(b0 + 2, set0)
            drain_a(set1)
            return carry
        lax.fori_loop(0, NB // 2, body_a, 0)

        plsc.subcore_barrier()

        if emit_cnt:
            pltpu.sync_copy(cnt_sp.at[pl.ds(sid * MSR, MSR)], cv1)
            pltpu.sync_copy(cv1, cnt_out.at[pl.ds(sid * MSR, MSR)])

        @pl.when(sid < MPD // MSR)
        def _scale():
            if not emit_cnt:
                pltpu.sync_copy(cnt_in.at[pl.ds(sid * MSR, MSR)], cv1)
            for r0 in range(0, MSR, K):
                nr = min(K, MSR - r0)
                e0 = sid * MSR + r0
                pltpu.sync_copy(buf.at[pl.ds(e0, nr)], rows_buf.at[pl.ds(0, nr)])

                def body_a2(m, carry, r0=r0):
                    cnt = plsc.load_gather(
                        cv1, [jnp.full((16,), r0 + m, jnp.int32)])
                    inv = 1.0 / jnp.maximum(cnt, 1.0)
                    for c in range(cpr):
                        sl = pl.ds(c * 16, 16)
                        rows_buf[m, sl] = rows_buf[m, sl] * inv
                    return carry
                lax.fori_loop(0, nr, body_a2, 0)
                pltpu.sync_copy(rows_buf.at[pl.ds(0, nr)],
                                out.at[pl.ds(off + e0, nr)])

        plsc.subcore_barrier()

        for r0 in range(0, NWR, K):
            nr = min(K, NWR - r0)
            a0 = sid * NWR + r0
            pltpu.sync_copy(hs.at[pl.ds(off + a0, nr)], rows_buf.at[pl.ds(0, nr)])
            pltpu.sync_copy(rows_buf.at[pl.ds(0, nr)], buf.at[pl.ds(a0, nr)])

        plsc.subcore_barrier()

        def fire_b(b, st):
            vb, ig, eb, rb, sem = st
            s0 = pl.multiple_of(base + b * K, 8)
            pltpu.sync_copy(vidx.at[pl.ds(s0, K)], vb)
            pltpu.sync_copy(eidx.at[pl.ds(s0, K)], eb)
            for i in range(K // 16):
                sl = pl.ds(i * 16, 16)
                ig[sl] = eb[sl] + off
            pltpu.async_copy(out.at[ig], rb, sem)

        def drain_b(st):
            vb, ig, eb, rb, sem = st
            pltpu.make_async_copy(out.at[pl.ds(0, K)], rb, sem).wait()
            pltpu.sync_copy(rb, buf.at[vb], add=True)

        fire_b(0, set0)

        def body_b(g, carry):
            b0 = g * 2
            fire_b(b0 + 1, set1)
            drain_b(set0)

            @pl.when(b0 + 2 < NB)
            def _():
                fire_b(b0 + 2, set0)
            drain_b(set1)
            return carry
        lax.fori_loop(0, NB // 2, body_b, 0)

        plsc.subcore_barrier()

        for r0 in range(0, NWR, K):
            nr = min(K, NWR - r0)
            a0 = sid * NWR + r0
            pltpu.sync_copy(buf.at[pl.ds(a0, nr)], rows_buf.at[pl.ds(0, nr)])
            pltpu.sync_copy(rows_buf.at[pl.ds(0, nr)], out.at[pl.ds(off + a0, nr)])

    return agg


_agg_emit = _make_agg(128, emit_cnt=True)
_agg_use = _make_agg(64, emit_cnt=False, tct=False)


def _tc1_body(x_ref, w_ref, b_ref, o_ref):
    h = lax.dot_general(x_ref[...], w_ref[...], (((1,), (1,)), ((), ())),
                        preferred_element_type=jnp.float32)
    h = h + b_ref[...]
    o_ref[0] = h[:, :128]
    o_ref[1] = h[:, 128:]


def _tc2_body(x0_ref, x1_ref, w_ref, b_ref, o_ref):
    x0 = jnp.maximum(x0_ref[0], 0.0)
    x1 = jnp.maximum(x1_ref[0], 0.0)
    w = w_ref[...]
    h = lax.dot_general(x0, w[:, :128], (((1,), (1,)), ((), ())),
                        preferred_element_type=jnp.float32)
    h = h + lax.dot_general(x1, w[:, 128:], (((1,), (1,)), ((), ())),
                            preferred_element_type=jnp.float32)
    h = h + b_ref[...]
    o_ref[0] = h
    o_ref[1] = h


_RB = 2528


def _tc1(x_pad, w1, b1r):
    return pl.pallas_call(
        _tc1_body,
        grid=(NP // _RB,),
        in_specs=[
            pl.BlockSpec((_RB, D_IN), lambda i: (i, 0)),
            pl.BlockSpec((D_HID, D_IN), lambda i: (0, 0)),
            pl.BlockSpec((1, D_HID), lambda i: (0, 0)),
        ],
        out_specs=pl.BlockSpec((2, _RB, 128), lambda i: (0, i, 0)),
        out_shape=jax.ShapeDtypeStruct((2, NP, 128), jnp.float32),
    )(x_pad, w1, b1r)


def _tc2(hs1, w2p, b2r):
    return pl.pallas_call(
        _tc2_body,
        grid=(NP // _RB,),
        in_specs=[
            pl.BlockSpec((1, _RB, 128), lambda i: (0, i, 0)),
            pl.BlockSpec((1, _RB, 128), lambda i: (1, i, 0)),
            pl.BlockSpec((64, D_HID), lambda i: (0, 0)),
            pl.BlockSpec((1, 64), lambda i: (0, 0)),
        ],
        out_specs=pl.BlockSpec((2, _RB, 64), lambda i: (0, i, 0)),
        out_shape=jax.ShapeDtypeStruct((2, NP, 64), jnp.float32),
    )(hs1, hs1, w2p, b2r)


def kernel(X, W1, b1, W2, b2, v_idx, e_idx):
    x_pad = jnp.pad(X, ((0, NP - N), (0, 0)))
    vpad = jnp.pad(v_idx, (0, NS * NBK - E), constant_values=NDUM)
    epad = jnp.pad(e_idx, (0, NS * NBK - E), constant_values=MDUM)
    h3 = _tc1(x_pad, W1, b1.reshape(1, -1))
    g1, cnt1 = _agg_emit(h3.reshape(2 * NP, 128), vpad, epad)
    w2p = jnp.pad(W2, ((0, 64 - N_CLS), (0, 0)))
    b2r = jnp.pad(b2, (0, 64 - N_CLS)).reshape(1, -1)
    h2 = _tc2(g1.reshape(2, NP, 128), w2p, b2r)
    g2 = _agg_use(h2.reshape(2 * NP, 64), vpad, epad, cnt1)
    return g2[:N, :N_CLS]


# speedup vs baseline: 1.5687x; 1.1144x over previous
import functools

import jax
import jax.numpy as jnp
from jax import lax
from jax.experimental import pallas as pl
from jax.experimental.pallas import tpu as pltpu
from jax.experimental.pallas import tpu_sc as plsc

N = 10000
M = 5000
E = 160000
D_IN = 256
D_HID = 256
N_CLS = 40

NC = 2
NS = 16
K = 160
NB = 64
NBK = NB * K

NP = 10112
MPD = 5376
MCT = 6144
NDUM = N
MDUM = 5120
NWR = 632
MZR = 336
MSR = 384


def _make_agg(DH: int, emit_cnt: bool, tct: bool = True):
    mesh = plsc.VectorSubcoreMesh(core_axis_name="c", subcore_axis_name="s")
    cpr = DH // 16

    out_sds = jax.ShapeDtypeStruct((2 * NP, DH), jnp.float32)
    cnt_sds = jax.ShapeDtypeStruct((MCT,), jnp.float32)
    scratch = [
        pltpu.VMEM((K,), jnp.int32),
        pltpu.VMEM((K,), jnp.int32),
        pltpu.VMEM((K,), jnp.int32),
        pltpu.VMEM((K, DH), jnp.float32),
        pltpu.VMEM((K,), jnp.int32),
        pltpu.VMEM((K,), jnp.int32),
        pltpu.VMEM((K,), jnp.int32),
        pltpu.VMEM((K, DH), jnp.float32),
        pltpu.VMEM((MSR,), jnp.float32),
        pltpu.VMEM((K,), jnp.float32),
        pltpu.VMEM_SHARED((NP, DH), jnp.float32),
        pltpu.VMEM_SHARED((MCT,), jnp.float32),
        pltpu.SemaphoreType.DMA,
        pltpu.SemaphoreType.DMA,
    ]
    if not emit_cnt:
        del scratch[11]

    @functools.partial(
        pl.kernel,
        out_type=(out_sds, cnt_sds) if emit_cnt else out_sds,
        mesh=mesh,
        compiler_params=pltpu.CompilerParams(
            needs_layout_passes=False, use_tc_tiling_on_sc=tct),
        scratch_types=scratch,
    )
    def agg(*refs):
        if emit_cnt:
            (hs, vidx, eidx, out, cnt_out, vb0, ig0, eb0, rb0, vb1, ig1, eb1,
             rb1, cv1, ones1, buf, cnt_sp, semA0, semA1) = refs
        else:
            (hs, vidx, eidx, cnt_in, out, vb0, ig0, eb0, rb0, vb1, ig1, eb1,
             rb1, cv1, ones1, buf, semA0, semA1) = refs
        rows_buf = rb0
        set0 = (vb0, ig0, eb0, rb0, semA0)
        set1 = (vb1, ig1, eb1, rb1, semA1)
        cid = lax.axis_index("c")
        sid = lax.axis_index("s")
        off = cid * NP
        base = pl.multiple_of(sid * NBK, 8)

        z16 = jnp.zeros((16,), jnp.float32)
        o16 = jnp.ones((16,), jnp.float32)

        def body_fill(i, carry):
            for c in range(cpr):
                rows_buf[i, pl.ds(c * 16, 16)] = z16
            return carry
        lax.fori_loop(0, K, body_fill, 0)

        if emit_cnt:
            def body_fill1(i, carry):
                ones1[pl.ds(i * 16, 16)] = o16
                return carry
            lax.fori_loop(0, K // 16, body_fill1, 0)

            def body_fill2(i, carry):
                cv1[pl.ds(i * 16, 16)] = z16
                return carry
            lax.fori_loop(0, MSR // 16, body_fill2, 0)

        for r0 in range(0, MZR, K):
            nr = min(K, MZR - r0)
            pltpu.sync_copy(rows_buf.at[pl.ds(0, nr)],
                            buf.at[pl.ds(sid * MZR + r0, nr)])
        if emit_cnt:
            pltpu.sync_copy(cv1, cnt_sp.at[pl.ds(sid * MSR, MSR)])

        plsc.subcore_barrier()

        def fire_a(b, st):
            vb, ig, eb, rb, sem = st
            s0 = pl.multiple_of(base + b * K, 8)
            pltpu.sync_copy(vidx.at[pl.ds(s0, K)], vb)
            pltpu.sync_copy(eidx.at[pl.ds(s0, K)], eb)
            for i in range(K // 16):
                sl = pl.ds(i * 16, 16)
                ig[sl] = vb[sl] + off
            pltpu.async_copy(hs.at[ig], rb, sem)

        def drain_a(st):
            vb, ig, eb, rb, sem = st
            pltpu.make_async_copy(hs.at[pl.ds(0, K)], rb, sem).wait()
            pltpu.sync_copy(rb, buf.at[eb], add=True)
            if emit_cnt:
                pltpu.sync_copy(ones1, cnt_sp.at[eb], add=True)

        fire_a(0, set0)

        def body_a(g, carry):
            b0 = g * 2
            fire_a(b0 + 1, set1)
            drain_a(set0)

            @pl.when(b0 + 2 < NB)
            def _():
                fire_a(b0 + 2, set0)
            drain_a(set1)
            return carry
        lax.fori_loop(0, NB // 2, body_a, 0)

        plsc.subcore_barrier()

        if emit_cnt:
            pltpu.sync_copy(cnt_sp.at[pl.ds(sid * MSR, MSR)], cv1)
            pltpu.sync_copy(cv1, cnt_out.at[pl.ds(sid * MSR, MSR)])

        @pl.when(sid < MPD // MSR)
        def _scale():
            if not emit_cnt:
                pltpu.sync_copy(cnt_in.at[pl.ds(sid * MSR, MSR)], cv1)
            for r0 in range(0, MSR, K):
                nr = min(K, MSR - r0)
                e0 = sid * MSR + r0
                pltpu.sync_copy(buf.at[pl.ds(e0, nr)], rows_buf.at[pl.ds(0, nr)])

                def body_a2(m, carry, r0=r0):
                    cnt = plsc.load_gather(
                        cv1, [jnp.full((16,), r0 + m, jnp.int32)])
                    inv = 1.0 / jnp.maximum(cnt, 1.0)
                    for c in range(cpr):
                        sl = pl.ds(c * 16, 16)
                        rows_buf[m, sl] = rows_buf[m, sl] * inv
                    return carry
                lax.fori_loop(0, nr, body_a2, 0)
                pltpu.sync_copy(rows_buf.at[pl.ds(0, nr)],
                                out.at[pl.ds(off + e0, nr)])

        plsc.subcore_barrier()

        for r0 in range(0, NWR, K):
            nr = min(K, NWR - r0)
            a0 = sid * NWR + r0
            pltpu.sync_copy(hs.at[pl.ds(off + a0, nr)], rows_buf.at[pl.ds(0, nr)])
            pltpu.sync_copy(rows_buf.at[pl.ds(0, nr)], buf.at[pl.ds(a0, nr)])

        plsc.subcore_barrier()

        def fire_b(b, st):
            vb, ig, eb, rb, sem = st
            s0 = pl.multiple_of(base + b * K, 8)
            pltpu.sync_copy(vidx.at[pl.ds(s0, K)], vb)
            pltpu.sync_copy(eidx.at[pl.ds(s0, K)], eb)
            for i in range(K // 16):
                sl = pl.ds(i * 16, 16)
                ig[sl] = eb[sl] + off
            pltpu.async_copy(out.at[ig], rb, sem)

        def drain_b(st):
            vb, ig, eb, rb, sem = st
            pltpu.make_async_copy(out.at[pl.ds(0, K)], rb, sem).wait()
            pltpu.sync_copy(rb, buf.at[vb], add=True)

        fire_b(0, set0)

        def body_b(g, carry):
            b0 = g * 2
            fire_b(b0 + 1, set1)
            drain_b(set0)

            @pl.when(b0 + 2 < NB)
            def _():
                fire_b(b0 + 2, set0)
            drain_b(set1)
            return carry
        lax.fori_loop(0, NB // 2, body_b, 0)

        plsc.subcore_barrier()

        for r0 in range(0, NWR, K):
            nr = min(K, NWR - r0)
            a0 = sid * NWR + r0
            pltpu.sync_copy(buf.at[pl.ds(a0, nr)], rows_buf.at[pl.ds(0, nr)])
            pltpu.sync_copy(rows_buf.at[pl.ds(0, nr)], out.at[pl.ds(off + a0, nr)])

    return agg


_agg_emit = _make_agg(128, emit_cnt=True)
_agg_use = _make_agg(32, emit_cnt=False, tct=False)


def _tc1_body(x_ref, w_ref, b_ref, o_ref):
    h = lax.dot_general(x_ref[...], w_ref[...], (((1,), (1,)), ((), ())),
                        preferred_element_type=jnp.float32)
    h = h + b_ref[...]
    o_ref[0] = h[:, :128]
    o_ref[1] = h[:, 128:]


def _tc2_body(x0_ref, x1_ref, w_ref, b_ref, o_ref):
    x0 = jnp.maximum(x0_ref[0], 0.0)
    x1 = jnp.maximum(x1_ref[0], 0.0)
    w = w_ref[...]
    h = lax.dot_general(x0, w[:, :128], (((1,), (1,)), ((), ())),
                        preferred_element_type=jnp.float32)
    h = h + lax.dot_general(x1, w[:, 128:], (((1,), (1,)), ((), ())),
                            preferred_element_type=jnp.float32)
    h = h + b_ref[...]
    o_ref[0] = h[:, :32]
    o_ref[1] = h[:, 32:]


_RB = 2528


def _tc1(x_pad, w1, b1r):
    return pl.pallas_call(
        _tc1_body,
        grid=(NP // _RB,),
        in_specs=[
            pl.BlockSpec((_RB, D_IN), lambda i: (i, 0)),
            pl.BlockSpec((D_HID, D_IN), lambda i: (0, 0)),
            pl.BlockSpec((1, D_HID), lambda i: (0, 0)),
        ],
        out_specs=pl.BlockSpec((2, _RB, 128), lambda i: (0, i, 0)),
        out_shape=jax.ShapeDtypeStruct((2, NP, 128), jnp.float32),
    )(x_pad, w1, b1r)


def _tc2(hs1, w2p, b2r):
    return pl.pallas_call(
        _tc2_body,
        grid=(NP // _RB,),
        in_specs=[
            pl.BlockSpec((1, _RB, 128), lambda i: (0, i, 0)),
            pl.BlockSpec((1, _RB, 128), lambda i: (1, i, 0)),
            pl.BlockSpec((64, D_HID), lambda i: (0, 0)),
            pl.BlockSpec((1, 64), lambda i: (0, 0)),
        ],
        out_specs=pl.BlockSpec((2, _RB, 32), lambda i: (0, i, 0)),
        out_shape=jax.ShapeDtypeStruct((2, NP, 32), jnp.float32),
    )(hs1, hs1, w2p, b2r)


def kernel(X, W1, b1, W2, b2, v_idx, e_idx):
    x_pad = jnp.pad(X, ((0, NP - N), (0, 0)))
    vpad = jnp.pad(v_idx, (0, NS * NBK - E), constant_values=NDUM)
    epad = jnp.pad(e_idx, (0, NS * NBK - E), constant_values=MDUM)
    h3 = _tc1(x_pad, W1, b1.reshape(1, -1))
    g1, cnt1 = _agg_emit(h3.reshape(2 * NP, 128), vpad, epad)
    w2p = jnp.pad(W2, ((0, 64 - N_CLS), (0, 0)))
    b2r = jnp.pad(b2, (0, 64 - N_CLS)).reshape(1, -1)
    h2 = _tc2(g1.reshape(2, NP, 128), w2p, b2r)
    g2 = _agg_use(h2.reshape(2 * NP, 32), vpad, epad, cnt1)
    return jnp.concatenate([g2[:N], g2[NP:NP + N, :N_CLS - 32]], axis=1)


# speedup vs baseline: 1.5780x; 1.0059x over previous
import functools

import jax
import jax.numpy as jnp
from jax import lax
from jax.experimental import pallas as pl
from jax.experimental.pallas import tpu as pltpu
from jax.experimental.pallas import tpu_sc as plsc

N = 10000
M = 5000
E = 160000
D_IN = 256
D_HID = 256
N_CLS = 40

NC = 2
NS = 16
K = 160
NB = 64
NBK = NB * K

NP = 10112
MPD = 5376
MCT = 6144
NDUM = N
MDUM = 5120
NWR = 632
MZR = 336
MSR = 384


def _make_agg(DH: int, emit_cnt: bool, tct: bool = True):
    mesh = plsc.VectorSubcoreMesh(core_axis_name="c", subcore_axis_name="s")
    cpr = DH // 16

    out_sds = jax.ShapeDtypeStruct((2 * NP, DH), jnp.float32)
    cnt_sds = jax.ShapeDtypeStruct((MCT,), jnp.float32)
    scratch = [
        pltpu.VMEM((K,), jnp.int32),
        pltpu.VMEM((K,), jnp.int32),
        pltpu.VMEM((K,), jnp.int32),
        pltpu.VMEM((K, DH), jnp.float32),
        pltpu.VMEM((K,), jnp.int32),
        pltpu.VMEM((K,), jnp.int32),
        pltpu.VMEM((K,), jnp.int32),
        pltpu.VMEM((K, DH), jnp.float32),
        pltpu.VMEM((MSR,), jnp.float32),
        pltpu.VMEM((K,), jnp.float32),
        pltpu.VMEM_SHARED((NP, DH), jnp.float32),
        pltpu.VMEM_SHARED((MCT,), jnp.float32),
        pltpu.SemaphoreType.DMA,
        pltpu.SemaphoreType.DMA,
    ]
    if not emit_cnt:
        del scratch[11]

    @functools.partial(
        pl.kernel,
        out_type=(out_sds, cnt_sds) if emit_cnt else out_sds,
        mesh=mesh,
        compiler_params=pltpu.CompilerParams(
            needs_layout_passes=False, use_tc_tiling_on_sc=tct),
        scratch_types=scratch,
    )
    def agg(*refs):
        if emit_cnt:
            (hs, vidx, eidx, out, cnt_out, vb0, ig0, eb0, rb0, vb1, ig1, eb1,
             rb1, cv1, ones1, buf, cnt_sp, semA0, semA1) = refs
        else:
            (hs, vidx, eidx, cnt_in, out, vb0, ig0, eb0, rb0, vb1, ig1, eb1,
             rb1, cv1, ones1, buf, semA0, semA1) = refs
        rows_buf = rb0
        set0 = (vb0, ig0, eb0, rb0, semA0)
        set1 = (vb1, ig1, eb1, rb1, semA1)
        cid = lax.axis_index("c")
        sid = lax.axis_index("s")
        off = cid * NP
        base = pl.multiple_of(sid * NBK, 8)

        z16 = jnp.zeros((16,), jnp.float32)
        o16 = jnp.ones((16,), jnp.float32)

        def body_fill(i, carry):
            for c in range(cpr):
                rows_buf[i, pl.ds(c * 16, 16)] = z16
            return carry
        lax.fori_loop(0, K, body_fill, 0)

        if emit_cnt:
            def body_fill1(i, carry):
                ones1[pl.ds(i * 16, 16)] = o16
                return carry
            lax.fori_loop(0, K // 16, body_fill1, 0)

            def body_fill2(i, carry):
                cv1[pl.ds(i * 16, 16)] = z16
                return carry
            lax.fori_loop(0, MSR // 16, body_fill2, 0)

        for r0 in range(0, MZR, K):
            nr = min(K, MZR - r0)
            pltpu.sync_copy(rows_buf.at[pl.ds(0, nr)],
                            buf.at[pl.ds(sid * MZR + r0, nr)])
        if emit_cnt:
            pltpu.sync_copy(cv1, cnt_sp.at[pl.ds(sid * MSR, MSR)])

        plsc.subcore_barrier()

        def fire_a(b, st):
            vb, ig, eb, rb, sem = st
            s0 = pl.multiple_of(base + b * K, 8)
            pltpu.sync_copy(vidx.at[pl.ds(s0, K)], vb)
            pltpu.sync_copy(eidx.at[pl.ds(s0, K)], eb)
            for i in range(K // 16):
                sl = pl.ds(i * 16, 16)
                ig[sl] = vb[sl] + off
            pltpu.async_copy(hs.at[ig], rb, sem)

        def drain_a(st):
            vb, ig, eb, rb, sem = st
            pltpu.make_async_copy(hs.at[pl.ds(0, K)], rb, sem).wait()
            pltpu.sync_copy(rb, buf.at[eb], add=True)
            if emit_cnt:
                pltpu.sync_copy(ones1, cnt_sp.at[eb], add=True)

        fire_a(0, set0)

        def body_a(g, carry):
            b0 = g * 2
            fire_a(b0 + 1, set1)
            drain_a(set0)

            @pl.when(b0 + 2 < NB)
            def _():
                fire_a(b0 + 2, set0)
            drain_a(set1)
            return carry
        lax.fori_loop(0, NB // 2, body_a, 0)

        plsc.subcore_barrier()

        if emit_cnt:
            pltpu.sync_copy(cnt_sp.at[pl.ds(sid * MSR, MSR)], cv1)
            pltpu.sync_copy(cv1, cnt_out.at[pl.ds(sid * MSR, MSR)])

        @pl.when(sid < MPD // MSR)
        def _scale():
            if not emit_cnt:
                pltpu.sync_copy(cnt_in.at[pl.ds(sid * MSR, MSR)], cv1)
            for r0 in range(0, MSR, K):
                nr = min(K, MSR - r0)
                e0 = sid * MSR + r0
                pltpu.sync_copy(buf.at[pl.ds(e0, nr)], rows_buf.at[pl.ds(0, nr)])

                def body_a2(m, carry, r0=r0):
                    cnt = plsc.load_gather(
                        cv1, [jnp.full((16,), r0 + m, jnp.int32)])
                    inv = 1.0 / jnp.maximum(cnt, 1.0)
                    for c in range(cpr):
                        sl = pl.ds(c * 16, 16)
                        rows_buf[m, sl] = rows_buf[m, sl] * inv
                    return carry
                lax.fori_loop(0, nr, body_a2, 0)
                pltpu.sync_copy(rows_buf.at[pl.ds(0, nr)],
                                out.at[pl.ds(off + e0, nr)])

        plsc.subcore_barrier()

        chunks = [(r0, min(K, NWR - r0)) for r0 in range(0, NWR, K)]
        prev = None
        for ci, (r0, nr) in enumerate(chunks):
            rb, sem = (rb0, semA0) if ci % 2 == 0 else (rb1, semA1)
            pltpu.async_copy(hs.at[pl.ds(off + sid * NWR + r0, nr)],
                             rb.at[pl.ds(0, nr)], sem)
            if prev is not None:
                pr0, pnr, prb, psem = prev
                pltpu.make_async_copy(hs.at[pl.ds(0, pnr)],
                                      prb.at[pl.ds(0, pnr)], psem).wait()
                pltpu.sync_copy(prb.at[pl.ds(0, pnr)],
                                buf.at[pl.ds(sid * NWR + pr0, pnr)])
            prev = (r0, nr, rb, sem)
        pr0, pnr, prb, psem = prev
        pltpu.make_async_copy(hs.at[pl.ds(0, pnr)],
                              prb.at[pl.ds(0, pnr)], psem).wait()
        pltpu.sync_copy(prb.at[pl.ds(0, pnr)],
                        buf.at[pl.ds(sid * NWR + pr0, pnr)])

        plsc.subcore_barrier()

        def fire_b(b, st):
            vb, ig, eb, rb, sem = st
            s0 = pl.multiple_of(base + b * K, 8)
            pltpu.sync_copy(vidx.at[pl.ds(s0, K)], vb)
            pltpu.sync_copy(eidx.at[pl.ds(s0, K)], eb)
            for i in range(K // 16):
                sl = pl.ds(i * 16, 16)
                ig[sl] = eb[sl] + off
            pltpu.async_copy(out.at[ig], rb, sem)

        def drain_b(st):
            vb, ig, eb, rb, sem = st
            pltpu.make_async_copy(out.at[pl.ds(0, K)], rb, sem).wait()
            pltpu.sync_copy(rb, buf.at[vb], add=True)

        fire_b(0, set0)

        def body_b(g, carry):
            b0 = g * 2
            fire_b(b0 + 1, set1)
            drain_b(set0)

            @pl.when(b0 + 2 < NB)
            def _():
                fire_b(b0 + 2, set0)
            drain_b(set1)
            return carry
        lax.fori_loop(0, NB // 2, body_b, 0)

        plsc.subcore_barrier()

        prev = None
        for ci, (r0, nr) in enumerate(chunks):
            rb, sem = (rb0, semA0) if ci % 2 == 0 else (rb1, semA1)
            pltpu.async_copy(buf.at[pl.ds(sid * NWR + r0, nr)],
                             rb.at[pl.ds(0, nr)], sem)
            if prev is not None:
                pr0, pnr, prb, psem = prev
                pltpu.make_async_copy(hs.at[pl.ds(0, pnr)],
                                      prb.at[pl.ds(0, pnr)], psem).wait()
                pltpu.sync_copy(prb.at[pl.ds(0, pnr)],
                                out.at[pl.ds(off + sid * NWR + pr0, pnr)])
            prev = (r0, nr, rb, sem)
        pr0, pnr, prb, psem = prev
        pltpu.make_async_copy(hs.at[pl.ds(0, pnr)],
                              prb.at[pl.ds(0, pnr)], psem).wait()
        pltpu.sync_copy(prb.at[pl.ds(0, pnr)],
                        out.at[pl.ds(off + sid * NWR + pr0, pnr)])

    return agg


_agg_emit = _make_agg(128, emit_cnt=True)
_agg_use = _make_agg(32, emit_cnt=False, tct=False)


def _tc1_body(x_ref, w_ref, b_ref, o_ref):
    h = lax.dot_general(x_ref[...], w_ref[...], (((1,), (1,)), ((), ())),
                        preferred_element_type=jnp.float32)
    h = h + b_ref[...]
    o_ref[0] = h[:, :128]
    o_ref[1] = h[:, 128:]


def _tc2_body(x0_ref, x1_ref, w_ref, b_ref, o_ref):
    x0 = jnp.maximum(x0_ref[0], 0.0)
    x1 = jnp.maximum(x1_ref[0], 0.0)
    w = w_ref[...]
    h = lax.dot_general(x0, w[:, :128], (((1,), (1,)), ((), ())),
                        preferred_element_type=jnp.float32)
    h = h + lax.dot_general(x1, w[:, 128:], (((1,), (1,)), ((), ())),
                            preferred_element_type=jnp.float32)
    h = h + b_ref[...]
    o_ref[0] = h[:, :32]
    o_ref[1] = h[:, 32:]


_RB = 2528


def _tc1(x_pad, w1, b1r):
    return pl.pallas_call(
        _tc1_body,
        grid=(NP // _RB,),
        in_specs=[
            pl.BlockSpec((_RB, D_IN), lambda i: (i, 0)),
            pl.BlockSpec((D_HID, D_IN), lambda i: (0, 0)),
            pl.BlockSpec((1, D_HID), lambda i: (0, 0)),
        ],
        out_specs=pl.BlockSpec((2, _RB, 128), lambda i: (0, i, 0)),
        out_shape=jax.ShapeDtypeStruct((2, NP, 128), jnp.float32),
    )(x_pad, w1, b1r)


def _tc2(hs1, w2p, b2r):
    return pl.pallas_call(
        _tc2_body,
        grid=(NP // _RB,),
        in_specs=[
            pl.BlockSpec((1, _RB, 128), lambda i: (0, i, 0)),
            pl.BlockSpec((1, _RB, 128), lambda i: (1, i, 0)),
            pl.BlockSpec((64, D_HID), lambda i: (0, 0)),
            pl.BlockSpec((1, 64), lambda i: (0, 0)),
        ],
        out_specs=pl.BlockSpec((2, _RB, 32), lambda i: (0, i, 0)),
        out_shape=jax.ShapeDtypeStruct((2, NP, 32), jnp.float32),
    )(hs1, hs1, w2p, b2r)


def kernel(X, W1, b1, W2, b2, v_idx, e_idx):
    x_pad = jnp.pad(X, ((0, NP - N), (0, 0)))
    vpad = jnp.pad(v_idx, (0, NS * NBK - E), constant_values=NDUM)
    epad = jnp.pad(e_idx, (0, NS * NBK - E), constant_values=MDUM)
    h3 = _tc1(x_pad, W1, b1.reshape(1, -1))
    g1, cnt1 = _agg_emit(h3.reshape(2 * NP, 128), vpad, epad)
    w2p = jnp.pad(W2, ((0, 64 - N_CLS), (0, 0)))
    b2r = jnp.pad(b2, (0, 64 - N_CLS)).reshape(1, -1)
    h2 = _tc2(g1.reshape(2, NP, 128), w2p, b2r)
    g2 = _agg_use(h2.reshape(2 * NP, 32), vpad, epad, cnt1)
    return jnp.concatenate([g2[:N], g2[NP:NP + N, :N_CLS - 32]], axis=1)


# speedup vs baseline: 1.5798x; 1.0012x over previous
import functools

import jax
import jax.numpy as jnp
from jax import lax
from jax.experimental import pallas as pl
from jax.experimental.pallas import tpu as pltpu
from jax.experimental.pallas import tpu_sc as plsc

N = 10000
M = 5000
E = 160000
D_IN = 256
D_HID = 256
N_CLS = 40

NC = 2
NS = 16
K = 160
NB = 64
NBK = NB * K

NP = 10112
MPD = 5376
MCT = 6144
NDUM = N
MDUM = 5120
NWR = 632
MZR = 336
MSR = 384


def _make_agg(DH: int, emit_cnt: bool, tct: bool = True):
    mesh = plsc.VectorSubcoreMesh(core_axis_name="c", subcore_axis_name="s")
    cpr = DH // 16

    out_sds = jax.ShapeDtypeStruct((2 * NP, DH), jnp.float32)
    cnt_sds = jax.ShapeDtypeStruct((MCT,), jnp.float32)
    scratch = [
        pltpu.VMEM((K,), jnp.int32),
        pltpu.VMEM((K,), jnp.int32),
        pltpu.VMEM((K,), jnp.int32),
        pltpu.VMEM((K, DH), jnp.float32),
        pltpu.VMEM((K,), jnp.int32),
        pltpu.VMEM((K,), jnp.int32),
        pltpu.VMEM((K,), jnp.int32),
        pltpu.VMEM((K, DH), jnp.float32),
        pltpu.VMEM((MSR,), jnp.float32),
        pltpu.VMEM((K,), jnp.float32),
        pltpu.VMEM_SHARED((NP, DH), jnp.float32),
        pltpu.VMEM_SHARED((MCT,), jnp.float32),
        pltpu.SemaphoreType.DMA,
        pltpu.SemaphoreType.DMA,
    ]
    if not emit_cnt:
        del scratch[11]

    @functools.partial(
        pl.kernel,
        out_type=(out_sds, cnt_sds) if emit_cnt else out_sds,
        mesh=mesh,
        compiler_params=pltpu.CompilerParams(
            needs_layout_passes=False, use_tc_tiling_on_sc=tct),
        scratch_types=scratch,
    )
    def agg(*refs):
        if emit_cnt:
            (hs, vidx, eidx, out, cnt_out, vb0, ig0, eb0, rb0, vb1, ig1, eb1,
             rb1, cv1, ones1, buf, cnt_sp, semA0, semA1) = refs
        else:
            (hs, vidx, eidx, cnt_in, out, vb0, ig0, eb0, rb0, vb1, ig1, eb1,
             rb1, cv1, ones1, buf, semA0, semA1) = refs
        rows_buf = rb0
        set0 = (vb0, ig0, eb0, rb0, semA0)
        set1 = (vb1, ig1, eb1, rb1, semA1)
        cid = lax.axis_index("c")
        sid = lax.axis_index("s")
        off = cid * NP
        base = pl.multiple_of(sid * NBK, 8)

        z16 = jnp.zeros((16,), jnp.float32)
        o16 = jnp.ones((16,), jnp.float32)

        def body_fill(i, carry):
            for c in range(cpr):
                rows_buf[i, pl.ds(c * 16, 16)] = z16
            return carry
        lax.fori_loop(0, K, body_fill, 0)

        if emit_cnt:
            def body_fill1(i, carry):
                ones1[pl.ds(i * 16, 16)] = o16
                return carry
            lax.fori_loop(0, K // 16, body_fill1, 0)

            def body_fill2(i, carry):
                cv1[pl.ds(i * 16, 16)] = z16
                return carry
            lax.fori_loop(0, MSR // 16, body_fill2, 0)

        for r0 in range(0, MZR, K):
            nr = min(K, MZR - r0)
            pltpu.sync_copy(rows_buf.at[pl.ds(0, nr)],
                            buf.at[pl.ds(sid * MZR + r0, nr)])
        if emit_cnt:
            pltpu.sync_copy(cv1, cnt_sp.at[pl.ds(sid * MSR, MSR)])

        plsc.subcore_barrier()

        def fire_a(b, st):
            vb, ig, eb, rb, sem = st
            s0 = pl.multiple_of(base + b * K, 8)
            pltpu.sync_copy(vidx.at[pl.ds(s0, K)], vb)
            pltpu.sync_copy(eidx.at[pl.ds(s0, K)], eb)
            for i in range(K // 16):
                sl = pl.ds(i * 16, 16)
                ig[sl] = vb[sl] + off
            pltpu.async_copy(hs.at[ig], rb, sem)

        def drain_a(st):
            vb, ig, eb, rb, sem = st
            pltpu.make_async_copy(hs.at[pl.ds(0, K)], rb, sem).wait()
            pltpu.sync_copy(rb, buf.at[eb], add=True)
            if emit_cnt:
                pltpu.sync_copy(ones1, cnt_sp.at[eb], add=True)

        fire_a(0, set0)

        def body_a(g, carry):
            b0 = g * 2
            fire_a(b0 + 1, set1)
            drain_a(set0)

            @pl.when(b0 + 2 < NB)
            def _():
                fire_a(b0 + 2, set0)
            drain_a(set1)
            return carry
        lax.fori_loop(0, NB // 2, body_a, 0)

        plsc.subcore_barrier()

        if emit_cnt:
            pltpu.sync_copy(cnt_sp.at[pl.ds(sid * MSR, MSR)], cv1)
            pltpu.sync_copy(cv1, cnt_out.at[pl.ds(sid * MSR, MSR)])

        @pl.when(sid < MPD // MSR)
        def _scale():
            if not emit_cnt:
                pltpu.sync_copy(cnt_in.at[pl.ds(sid * MSR, MSR)], cv1)
            for r0 in range(0, MSR, K):
                nr = min(K, MSR - r0)
                e0 = sid * MSR + r0
                pltpu.sync_copy(buf.at[pl.ds(e0, nr)], rows_buf.at[pl.ds(0, nr)])

                def body_a2(m, carry, r0=r0):
                    cnt = plsc.load_gather(
                        cv1, [jnp.full((16,), r0 + m, jnp.int32)])
                    inv = 1.0 / jnp.maximum(cnt, 1.0)
                    for c in range(cpr):
                        sl = pl.ds(c * 16, 16)
                        rows_buf[m, sl] = rows_buf[m, sl] * inv
                    return carry
                lax.fori_loop(0, nr, body_a2, 0)
                pltpu.sync_copy(rows_buf.at[pl.ds(0, nr)],
                                out.at[pl.ds(off + e0, nr)])

        plsc.subcore_barrier()

        chunks = [(r0, min(K, NWR - r0)) for r0 in range(0, NWR, K)]
        prev = None
        for ci, (r0, nr) in enumerate(chunks):
            rb, sem = (rb0, semA0) if ci % 2 == 0 else (rb1, semA1)
            pltpu.async_copy(hs.at[pl.ds(off + sid * NWR + r0, nr)],
                             rb.at[pl.ds(0, nr)], sem)
            if prev is not None:
                pr0, pnr, prb, psem = prev
                pltpu.make_async_copy(hs.at[pl.ds(0, pnr)],
                                      prb.at[pl.ds(0, pnr)], psem).wait()
                pltpu.sync_copy(prb.at[pl.ds(0, pnr)],
                                buf.at[pl.ds(sid * NWR + pr0, pnr)])
            prev = (r0, nr, rb, sem)
        pr0, pnr, prb, psem = prev
        pltpu.make_async_copy(hs.at[pl.ds(0, pnr)],
                              prb.at[pl.ds(0, pnr)], psem).wait()
        pltpu.sync_copy(prb.at[pl.ds(0, pnr)],
                        buf.at[pl.ds(sid * NWR + pr0, pnr)])

        plsc.subcore_barrier()

        def fire_b(b, st):
            vb, ig, eb, rb, sem = st
            s0 = pl.multiple_of(base + b * K, 8)
            pltpu.sync_copy(vidx.at[pl.ds(s0, K)], vb)
            pltpu.sync_copy(eidx.at[pl.ds(s0, K)], eb)
            for i in range(K // 16):
                sl = pl.ds(i * 16, 16)
                ig[sl] = eb[sl] + off
            pltpu.async_copy(out.at[ig], rb, sem)

        def drain_b(st):
            vb, ig, eb, rb, sem = st
            pltpu.make_async_copy(out.at[pl.ds(0, K)], rb, sem).wait()
            pltpu.sync_copy(rb, buf.at[vb], add=True)

        fire_b(0, set0)

        def body_b(g, carry):
            b0 = g * 2
            fire_b(b0 + 1, set1)
            drain_b(set0)

            @pl.when(b0 + 2 < NB)
            def _():
                fire_b(b0 + 2, set0)
            drain_b(set1)
            return carry
        lax.fori_loop(0, NB // 2, body_b, 0)

        plsc.subcore_barrier()

        prev = None
        for ci, (r0, nr) in enumerate(chunks):
            rb, sem = (rb0, semA0) if ci % 2 == 0 else (rb1, semA1)
            pltpu.async_copy(buf.at[pl.ds(sid * NWR + r0, nr)],
                             rb.at[pl.ds(0, nr)], sem)
            if prev is not None:
                pr0, pnr, prb, psem = prev
                pltpu.make_async_copy(hs.at[pl.ds(0, pnr)],
                                      prb.at[pl.ds(0, pnr)], psem).wait()
                pltpu.sync_copy(prb.at[pl.ds(0, pnr)],
                                out.at[pl.ds(off + sid * NWR + pr0, pnr)])
            prev = (r0, nr, rb, sem)
        pr0, pnr, prb, psem = prev
        pltpu.make_async_copy(hs.at[pl.ds(0, pnr)],
                              prb.at[pl.ds(0, pnr)], psem).wait()
        pltpu.sync_copy(prb.at[pl.ds(0, pnr)],
                        out.at[pl.ds(off + sid * NWR + pr0, pnr)])

    return agg


_agg_emit = _make_agg(128, emit_cnt=True, tct=False)
_agg_use = _make_agg(32, emit_cnt=False, tct=False)


def _tc1_body(x_ref, w_ref, b_ref, o_ref):
    h = lax.dot_general(x_ref[...], w_ref[...], (((1,), (1,)), ((), ())),
                        preferred_element_type=jnp.float32)
    h = h + b_ref[...]
    o_ref[0] = h[:, :128]
    o_ref[1] = h[:, 128:]


def _tc2_body(x0_ref, x1_ref, w_ref, b_ref, o_ref):
    x0 = jnp.maximum(x0_ref[0], 0.0)
    x1 = jnp.maximum(x1_ref[0], 0.0)
    w = w_ref[...]
    h = lax.dot_general(x0, w[:, :128], (((1,), (1,)), ((), ())),
                        preferred_element_type=jnp.float32)
    h = h + lax.dot_general(x1, w[:, 128:], (((1,), (1,)), ((), ())),
                            preferred_element_type=jnp.float32)
    h = h + b_ref[...]
    o_ref[0] = h[:, :32]
    o_ref[1] = h[:, 32:]


_RB = 2528


def _tc1(x_pad, w1, b1r):
    return pl.pallas_call(
        _tc1_body,
        grid=(NP // _RB,),
        in_specs=[
            pl.BlockSpec((_RB, D_IN), lambda i: (i, 0)),
            pl.BlockSpec((D_HID, D_IN), lambda i: (0, 0)),
            pl.BlockSpec((1, D_HID), lambda i: (0, 0)),
        ],
        out_specs=pl.BlockSpec((2, _RB, 128), lambda i: (0, i, 0)),
        out_shape=jax.ShapeDtypeStruct((2, NP, 128), jnp.float32),
    )(x_pad, w1, b1r)


def _tc2(hs1, w2p, b2r):
    return pl.pallas_call(
        _tc2_body,
        grid=(NP // _RB,),
        in_specs=[
            pl.BlockSpec((1, _RB, 128), lambda i: (0, i, 0)),
            pl.BlockSpec((1, _RB, 128), lambda i: (1, i, 0)),
            pl.BlockSpec((64, D_HID), lambda i: (0, 0)),
            pl.BlockSpec((1, 64), lambda i: (0, 0)),
        ],
        out_specs=pl.BlockSpec((2, _RB, 32), lambda i: (0, i, 0)),
        out_shape=jax.ShapeDtypeStruct((2, NP, 32), jnp.float32),
    )(hs1, hs1, w2p, b2r)


def kernel(X, W1, b1, W2, b2, v_idx, e_idx):
    x_pad = jnp.pad(X, ((0, NP - N), (0, 0)))
    vpad = jnp.pad(v_idx, (0, NS * NBK - E), constant_values=NDUM)
    epad = jnp.pad(e_idx, (0, NS * NBK - E), constant_values=MDUM)
    h3 = _tc1(x_pad, W1, b1.reshape(1, -1))
    g1, cnt1 = _agg_emit(h3.reshape(2 * NP, 128), vpad, epad)
    w2p = jnp.pad(W2, ((0, 64 - N_CLS), (0, 0)))
    b2r = jnp.pad(b2, (0, 64 - N_CLS)).reshape(1, -1)
    h2 = _tc2(g1.reshape(2, NP, 128), w2p, b2r)
    g2 = _agg_use(h2.reshape(2 * NP, 32), vpad, epad, cnt1)
    return jnp.concatenate([g2[:N], g2[NP:NP + N, :N_CLS - 32]], axis=1)


# speedup vs baseline: 1.6664x; 1.0548x over previous
import functools

import jax
import jax.numpy as jnp
from jax import lax
from jax.experimental import pallas as pl
from jax.experimental.pallas import tpu as pltpu
from jax.experimental.pallas import tpu_sc as plsc

N = 10000
M = 5000
E = 160000
D_IN = 256
D_HID = 256
N_CLS = 40

NC = 2
NS = 16
K = 160
NB = 64
NBK = NB * K

NP = 10112
MPD = 5376
MCT = 6144
NDUM = N
MDUM = 5120
NWR = 632
MZR = 336
MSR = 384


def _make_agg(DH: int, emit_cnt: bool, tct: bool = True, k: int = K):
    K = k
    NB = NBK // k
    mesh = plsc.VectorSubcoreMesh(core_axis_name="c", subcore_axis_name="s")
    cpr = DH // 16

    out_sds = jax.ShapeDtypeStruct((2 * NP, DH), jnp.float32)
    cnt_sds = jax.ShapeDtypeStruct((MCT,), jnp.float32)
    scratch = [
        pltpu.VMEM((K,), jnp.int32),
        pltpu.VMEM((K,), jnp.int32),
        pltpu.VMEM((K,), jnp.int32),
        pltpu.VMEM((K, DH), jnp.float32),
        pltpu.VMEM((K,), jnp.int32),
        pltpu.VMEM((K,), jnp.int32),
        pltpu.VMEM((K,), jnp.int32),
        pltpu.VMEM((K, DH), jnp.float32),
        pltpu.VMEM((MSR,), jnp.float32),
        pltpu.VMEM((K,), jnp.float32),
        pltpu.VMEM_SHARED((NP, DH), jnp.float32),
        pltpu.VMEM_SHARED((MCT,), jnp.float32),
        pltpu.SemaphoreType.DMA,
        pltpu.SemaphoreType.DMA,
    ]
    if not emit_cnt:
        del scratch[11]

    @functools.partial(
        pl.kernel,
        out_type=(out_sds, cnt_sds) if emit_cnt else out_sds,
        mesh=mesh,
        compiler_params=pltpu.CompilerParams(
            needs_layout_passes=False, use_tc_tiling_on_sc=tct),
        scratch_types=scratch,
    )
    def agg(*refs):
        if emit_cnt:
            (hs, vidx, eidx, out, cnt_out, vb0, ig0, eb0, rb0, vb1, ig1, eb1,
             rb1, cv1, ones1, buf, cnt_sp, semA0, semA1) = refs
        else:
            (hs, vidx, eidx, cnt_in, out, vb0, ig0, eb0, rb0, vb1, ig1, eb1,
             rb1, cv1, ones1, buf, semA0, semA1) = refs
        rows_buf = rb0
        set0 = (vb0, ig0, eb0, rb0, semA0)
        set1 = (vb1, ig1, eb1, rb1, semA1)
        cid = lax.axis_index("c")
        sid = lax.axis_index("s")
        off = cid * NP
        base = pl.multiple_of(sid * NBK, 8)

        z16 = jnp.zeros((16,), jnp.float32)
        o16 = jnp.ones((16,), jnp.float32)

        def body_fill(i, carry):
            for c in range(cpr):
                rows_buf[i, pl.ds(c * 16, 16)] = z16
            return carry
        lax.fori_loop(0, K, body_fill, 0)

        if emit_cnt:
            def body_fill1(i, carry):
                ones1[pl.ds(i * 16, 16)] = o16
                return carry
            lax.fori_loop(0, K // 16, body_fill1, 0)

            def body_fill2(i, carry):
                cv1[pl.ds(i * 16, 16)] = z16
                return carry
            lax.fori_loop(0, MSR // 16, body_fill2, 0)

        for r0 in range(0, MZR, K):
            nr = min(K, MZR - r0)
            pltpu.sync_copy(rows_buf.at[pl.ds(0, nr)],
                            buf.at[pl.ds(sid * MZR + r0, nr)])
        if emit_cnt:
            pltpu.sync_copy(cv1, cnt_sp.at[pl.ds(sid * MSR, MSR)])

        plsc.subcore_barrier()

        def fire_a(b, st):
            vb, ig, eb, rb, sem = st
            s0 = pl.multiple_of(base + b * K, 8)
            pltpu.sync_copy(vidx.at[pl.ds(s0, K)], vb)
            pltpu.sync_copy(eidx.at[pl.ds(s0, K)], eb)
            for i in range(K // 16):
                sl = pl.ds(i * 16, 16)
                ig[sl] = vb[sl] + off
            pltpu.async_copy(hs.at[ig], rb, sem)

        def drain_a(st):
            vb, ig, eb, rb, sem = st
            pltpu.make_async_copy(hs.at[pl.ds(0, K)], rb, sem).wait()
            pltpu.sync_copy(rb, buf.at[eb], add=True)
            if emit_cnt:
                pltpu.sync_copy(ones1, cnt_sp.at[eb], add=True)

        fire_a(0, set0)

        def body_a(g, carry):
            b0 = g * 2
            fire_a(b0 + 1, set1)
            drain_a(set0)

            @pl.when(b0 + 2 < NB)
            def _():
                fire_a(b0 + 2, set0)
            drain_a(set1)
            return carry
        lax.fori_loop(0, NB // 2, body_a, 0)

        plsc.subcore_barrier()

        if emit_cnt:
            pltpu.sync_copy(cnt_sp.at[pl.ds(sid * MSR, MSR)], cv1)
            pltpu.sync_copy(cv1, cnt_out.at[pl.ds(sid * MSR, MSR)])

        @pl.when(sid < MPD // MSR)
        def _scale():
            if not emit_cnt:
                pltpu.sync_copy(cnt_in.at[pl.ds(sid * MSR, MSR)], cv1)
            for r0 in range(0, MSR, K):
                nr = min(K, MSR - r0)
                e0 = sid * MSR + r0
                pltpu.sync_copy(buf.at[pl.ds(e0, nr)], rows_buf.at[pl.ds(0, nr)])

                def body_a2(m, carry, r0=r0):
                    cnt = plsc.load_gather(
                        cv1, [jnp.full((16,), r0 + m, jnp.int32)])
                    inv = 1.0 / jnp.maximum(cnt, 1.0)
                    for c in range(cpr):
                        sl = pl.ds(c * 16, 16)
                        rows_buf[m, sl] = rows_buf[m, sl] * inv
                    return carry
                lax.fori_loop(0, nr, body_a2, 0)
                pltpu.sync_copy(rows_buf.at[pl.ds(0, nr)],
                                out.at[pl.ds(off + e0, nr)])

        plsc.subcore_barrier()

        chunks = [(r0, min(K, NWR - r0)) for r0 in range(0, NWR, K)]
        prev = None
        for ci, (r0, nr) in enumerate(chunks):
            rb, sem = (rb0, semA0) if ci % 2 == 0 else (rb1, semA1)
            pltpu.async_copy(hs.at[pl.ds(off + sid * NWR + r0, nr)],
                             rb.at[pl.ds(0, nr)], sem)
            if prev is not None:
                pr0, pnr, prb, psem = prev
                pltpu.make_async_copy(hs.at[pl.ds(0, pnr)],
                                      prb.at[pl.ds(0, pnr)], psem).wait()
                pltpu.sync_copy(prb.at[pl.ds(0, pnr)],
                                buf.at[pl.ds(sid * NWR + pr0, pnr)])
            prev = (r0, nr, rb, sem)
        pr0, pnr, prb, psem = prev
        pltpu.make_async_copy(hs.at[pl.ds(0, pnr)],
                              prb.at[pl.ds(0, pnr)], psem).wait()
        pltpu.sync_copy(prb.at[pl.ds(0, pnr)],
                        buf.at[pl.ds(sid * NWR + pr0, pnr)])

        plsc.subcore_barrier()

        def fire_b(b, st):
            vb, ig, eb, rb, sem = st
            s0 = pl.multiple_of(base + b * K, 8)
            pltpu.sync_copy(vidx.at[pl.ds(s0, K)], vb)
            pltpu.sync_copy(eidx.at[pl.ds(s0, K)], eb)
            for i in range(K // 16):
                sl = pl.ds(i * 16, 16)
                ig[sl] = eb[sl] + off
            pltpu.async_copy(out.at[ig], rb, sem)

        def drain_b(st):
            vb, ig, eb, rb, sem = st
            pltpu.make_async_copy(out.at[pl.ds(0, K)], rb, sem).wait()
            pltpu.sync_copy(rb, buf.at[vb], add=True)

        fire_b(0, set0)

        def body_b(g, carry):
            b0 = g * 2
            fire_b(b0 + 1, set1)
            drain_b(set0)

            @pl.when(b0 + 2 < NB)
            def _():
                fire_b(b0 + 2, set0)
            drain_b(set1)
            return carry
        lax.fori_loop(0, NB // 2, body_b, 0)

        plsc.subcore_barrier()

        prev = None
        for ci, (r0, nr) in enumerate(chunks):
            rb, sem = (rb0, semA0) if ci % 2 == 0 else (rb1, semA1)
            pltpu.async_copy(buf.at[pl.ds(sid * NWR + r0, nr)],
                             rb.at[pl.ds(0, nr)], sem)
            if prev is not None:
                pr0, pnr, prb, psem = prev
                pltpu.make_async_copy(hs.at[pl.ds(0, pnr)],
                                      prb.at[pl.ds(0, pnr)], psem).wait()
                pltpu.sync_copy(prb.at[pl.ds(0, pnr)],
                                out.at[pl.ds(off + sid * NWR + pr0, pnr)])
            prev = (r0, nr, rb, sem)
        pr0, pnr, prb, psem = prev
        pltpu.make_async_copy(hs.at[pl.ds(0, pnr)],
                              prb.at[pl.ds(0, pnr)], psem).wait()
        pltpu.sync_copy(prb.at[pl.ds(0, pnr)],
                        out.at[pl.ds(off + sid * NWR + pr0, pnr)])

    return agg


_agg_emit = _make_agg(128, emit_cnt=True, tct=False)
_agg_use = _make_agg(32, emit_cnt=False, tct=False, k=512)


def _tc1_body(x_ref, w_ref, b_ref, o_ref):
    h = lax.dot_general(x_ref[...], w_ref[...], (((1,), (1,)), ((), ())),
                        preferred_element_type=jnp.float32)
    h = h + b_ref[...]
    o_ref[0] = h[:, :128]
    o_ref[1] = h[:, 128:]


def _tc2_body(x0_ref, x1_ref, w_ref, b_ref, o_ref):
    x0 = jnp.maximum(x0_ref[0], 0.0)
    x1 = jnp.maximum(x1_ref[0], 0.0)
    w = w_ref[...]
    h = lax.dot_general(x0, w[:, :128], (((1,), (1,)), ((), ())),
                        preferred_element_type=jnp.float32)
    h = h + lax.dot_general(x1, w[:, 128:], (((1,), (1,)), ((), ())),
                            preferred_element_type=jnp.float32)
    h = h + b_ref[...]
    o_ref[0] = h[:, :32]
    o_ref[1] = h[:, 32:]


_RB = 2528


def _tc1(x_pad, w1, b1r):
    return pl.pallas_call(
        _tc1_body,
        grid=(NP // _RB,),
        in_specs=[
            pl.BlockSpec((_RB, D_IN), lambda i: (i, 0)),
            pl.BlockSpec((D_HID, D_IN), lambda i: (0, 0)),
            pl.BlockSpec((1, D_HID), lambda i: (0, 0)),
        ],
        out_specs=pl.BlockSpec((2, _RB, 128), lambda i: (0, i, 0)),
        out_shape=jax.ShapeDtypeStruct((2, NP, 128), jnp.float32),
    )(x_pad, w1, b1r)


def _tc2(hs1, w2p, b2r):
    return pl.pallas_call(
        _tc2_body,
        grid=(NP // _RB,),
        in_specs=[
            pl.BlockSpec((1, _RB, 128), lambda i: (0, i, 0)),
            pl.BlockSpec((1, _RB, 128), lambda i: (1, i, 0)),
            pl.BlockSpec((64, D_HID), lambda i: (0, 0)),
            pl.BlockSpec((1, 64), lambda i: (0, 0)),
        ],
        out_specs=pl.BlockSpec((2, _RB, 32), lambda i: (0, i, 0)),
        out_shape=jax.ShapeDtypeStruct((2, NP, 32), jnp.float32),
    )(hs1, hs1, w2p, b2r)


def kernel(X, W1, b1, W2, b2, v_idx, e_idx):
    x_pad = jnp.pad(X, ((0, NP - N), (0, 0)))
    vpad = jnp.pad(v_idx, (0, NS * NBK - E), constant_values=NDUM)
    epad = jnp.pad(e_idx, (0, NS * NBK - E), constant_values=MDUM)
    h3 = _tc1(x_pad, W1, b1.reshape(1, -1))
    g1, cnt1 = _agg_emit(h3.reshape(2 * NP, 128), vpad, epad)
    w2p = jnp.pad(W2, ((0, 64 - N_CLS), (0, 0)))
    b2r = jnp.pad(b2, (0, 64 - N_CLS)).reshape(1, -1)
    h2 = _tc2(g1.reshape(2, NP, 128), w2p, b2r)
    g2 = _agg_use(h2.reshape(2 * NP, 32), vpad, epad, cnt1)
    return jnp.concatenate([g2[:N], g2[NP:NP + N, :N_CLS - 32]], axis=1)
